# baseline (speedup 1.0000x reference)
import functools

import jax
import jax.numpy as jnp
from jax.experimental import pallas as pl
from jax.experimental.pallas import tpu as pltpu

B = 1024
EMB = 32
MOVIES = 100001
K = 100

_COLS = 100352


def _scores_kernel(q_ref, ct_ref, o_ref):
    j = pl.program_id(0)
    s = jnp.dot(q_ref[...], ct_ref[...], preferred_element_type=jnp.float32)
    col = j * 512 + jax.lax.broadcasted_iota(jnp.int32, s.shape, 1)
    o_ref[...] = jnp.where(col < MOVIES, s, -jnp.inf)


def _scores(q, ct):
    return pl.pallas_call(
        _scores_kernel,
        grid=(_COLS // 512,),
        in_specs=[
            pl.BlockSpec((B, EMB), lambda j: (0, 0)),
            pl.BlockSpec((EMB, 512), lambda j: (0, j)),
        ],
        out_specs=pl.BlockSpec((B, 512), lambda j: (0, j)),
        out_shape=jax.ShapeDtypeStruct((B, _COLS), jnp.float32),
    )(q, ct)


def _masked_avg(table, ids):
    emb = jnp.take(table, ids, axis=0)
    mask = (ids != 0).astype(jnp.float32)[..., None]
    s = jnp.sum(emb * mask, axis=1)
    cnt = jnp.maximum(jnp.sum(mask, axis=1), 1.0)
    return s / cnt


def kernel(user_id, user_gender, raw_user_age, user_gender_X_raw_user_age, user_occupation_label, movie_id, movie_title_vector, movie_genres, user_table, gender_table, age_table, cross_table, occ_table, Wq, bq, movie_table, title_table, genre_table, Wc, bc, movies_title_vector, movies_genres):
    q_in = jnp.concatenate([
        jnp.take(user_table, user_id, axis=0),
        jnp.take(gender_table, user_gender, axis=0),
        jnp.take(age_table, raw_user_age, axis=0),
        jnp.take(cross_table, user_gender_X_raw_user_age, axis=0),
        jnp.take(occ_table, user_occupation_label, axis=0),
    ], axis=1)
    query_embeddings = q_in @ Wq + bq
    c_in = jnp.concatenate([
        jnp.take(movie_table, movie_id, axis=0),
        _masked_avg(title_table, movie_title_vector),
        _masked_avg(genre_table, movie_genres),
    ], axis=1)
    candidate_embeddings = c_in @ Wc + bc
    all_in = jnp.concatenate([
        movie_table,
        _masked_avg(title_table, movies_title_vector),
        _masked_avg(genre_table, movies_genres),
    ], axis=1)
    all_candidates = all_in @ Wc + bc
    ct = jnp.pad(all_candidates, ((0, _COLS - MOVIES), (0, 0))).T
    scores = _scores(query_embeddings, ct)
    _, predictions = jax.lax.top_k(scores, K)
    predictions = predictions.astype(jnp.int32)
    return (query_embeddings, candidate_embeddings, predictions)


# speedup vs baseline: 1.1722x; 1.1722x over previous
import functools

import jax
import jax.numpy as jnp
from jax import lax
from jax.experimental import pallas as pl
from jax.experimental.pallas import tpu as pltpu
from jax.experimental.pallas import tpu_sc as plsc

B = 1024
EMB = 32
MOVIES = 100001
TITLES = 10001
GENRES = 21
K = 100

_COLS = 100352
_ROWS_ALL = _COLS + B
_NW = 32
_MPW = _ROWS_ALL // _NW
_TCH = 32
_TNCH = _MPW // _TCH
_GCH = 96
_GNCH = _MPW // _GCH
_NEG = -3.4e38


def _sc_gather(title_aug, tids_flat, genre_aug, gids_flat,
               user_table, user_id, movie_table, movie_id):
    mesh = plsc.VectorSubcoreMesh(core_axis_name="c", subcore_axis_name="s")

    @functools.partial(
        pl.kernel,
        mesh=mesh,
        compiler_params=pltpu.CompilerParams(use_tc_tiling_on_sc=False),
        out_type=(
            jax.ShapeDtypeStruct((_ROWS_ALL, EMB), jnp.float32),
            jax.ShapeDtypeStruct((_ROWS_ALL, EMB), jnp.float32),
            jax.ShapeDtypeStruct((B, EMB), jnp.float32),
            jax.ShapeDtypeStruct((B, EMB), jnp.float32),
        ),
        scratch_types=[
            pltpu.VMEM((4, 128), jnp.int32),
            pltpu.VMEM((6, 96), jnp.int32),
            pltpu.VMEM((576, EMB), jnp.float32),
            pltpu.VMEM((_GCH, EMB), jnp.float32),
            pltpu.VMEM((32,), jnp.int32),
            pltpu.VMEM((32, EMB), jnp.float32),
            pltpu.SemaphoreType.DMA,
        ],
    )
    def k(title_hbm, tids_hbm, genre_hbm, gids_hbm, ut_hbm, uid_hbm,
          mt_hbm, mid_hbm,
          tsums_hbm, gsums_hbm, urows_hbm, mrows_hbm,
          idx_v, gidx_v, rows_v, osum_v, sidx_v, srows_v, sem):
        wid = lax.axis_index("s") * 2 + lax.axis_index("c")
        mbase = wid * _MPW

        def tchunk(c, carry):
            m0 = mbase + c * _TCH
            i0 = m0 * 16
            for j in range(4):
                pltpu.sync_copy(tids_hbm.at[pl.ds(i0 + j * 128, 128)],
                                idx_v.at[j])
            for j in range(4):
                for t in range(8):
                    v = idx_v[j, pl.ds(t * 16, 16)]
                    idx_v[j, pl.ds(t * 16, 16)] = jnp.where(
                        v == 0, jnp.full((16,), TITLES, jnp.int32), v)
            hs = []
            for j in range(4):
                hs.append(pltpu.async_copy(
                    title_hbm.at[idx_v.at[j]],
                    rows_v.at[pl.ds(j * 128, 128)], sem))
            for h in hs:
                h.wait()
            for m in range(_TCH):
                a0 = rows_v[m * 16, pl.ds(0, 16)]
                a1 = rows_v[m * 16, pl.ds(16, 16)]
                for t in range(1, 16):
                    a0 = a0 + rows_v[m * 16 + t, pl.ds(0, 16)]
                    a1 = a1 + rows_v[m * 16 + t, pl.ds(16, 16)]
                osum_v[m, pl.ds(0, 16)] = a0
                osum_v[m, pl.ds(16, 16)] = a1
            pltpu.sync_copy(osum_v.at[pl.ds(0, _TCH)],
                            tsums_hbm.at[pl.ds(m0, _TCH)])
            return carry

        lax.fori_loop(0, _TNCH, tchunk, 0)

        def gchunk(c, carry):
            m0 = mbase + c * _GCH
            i0 = m0 * 6
            for j in range(6):
                pltpu.sync_copy(gids_hbm.at[pl.ds(i0 + j * 96, 96)],
                                gidx_v.at[j])
            for j in range(6):
                for t in range(6):
                    v = gidx_v[j, pl.ds(t * 16, 16)]
                    gidx_v[j, pl.ds(t * 16, 16)] = jnp.where(
                        v == 0, jnp.full((16,), GENRES, jnp.int32), v)
            hs = []
            for j in range(6):
                hs.append(pltpu.async_copy(
                    genre_hbm.at[gidx_v.at[j]],
                    rows_v.at[pl.ds(j * 96, 96)], sem))
            for h in hs:
                h.wait()
            for m in range(_GCH):
                a0 = rows_v[m * 6, pl.ds(0, 16)]
                a1 = rows_v[m * 6, pl.ds(16, 16)]
                for t in range(1, 6):
                    a0 = a0 + rows_v[m * 6 + t, pl.ds(0, 16)]
                    a1 = a1 + rows_v[m * 6 + t, pl.ds(16, 16)]
                osum_v[m, pl.ds(0, 16)] = a0
                osum_v[m, pl.ds(16, 16)] = a1
            pltpu.sync_copy(osum_v, gsums_hbm.at[pl.ds(m0, _GCH)])
            return carry

        lax.fori_loop(0, _GNCH, gchunk, 0)

        rbase = wid * 32
        pltpu.sync_copy(uid_hbm.at[pl.ds(rbase, 32)], sidx_v)
        pltpu.async_copy(ut_hbm.at[sidx_v], srows_v, sem).wait()
        pltpu.sync_copy(srows_v, urows_hbm.at[pl.ds(rbase, 32)])
        pltpu.sync_copy(mid_hbm.at[pl.ds(rbase, 32)], sidx_v)
        pltpu.async_copy(mt_hbm.at[sidx_v], srows_v, sem).wait()
        pltpu.sync_copy(srows_v, mrows_hbm.at[pl.ds(rbase, 32)])

    return k(title_aug, tids_flat, genre_aug, gids_flat,
             user_table, user_id, movie_table, movie_id)


def _towers_kernel(urows_ref, g_ref, a_ref, c_ref, o_ref,
                   gt_ref, at_ref, ct_ref, ot_ref,
                   wq_ref, bq_ref,
                   mrows_ref, tsb_ref, gsb_ref, tvb_ref, gvb_ref,
                   wc_ref, bc_ref, qe_ref, ce_ref):
    def lut(ids_ref, tab_ref):
        ids = ids_ref[...]
        iota = lax.broadcasted_iota(jnp.int32, (B, 32), 1)
        oh = (ids == iota).astype(jnp.float32)
        return jnp.dot(oh, tab_ref[...], preferred_element_type=jnp.float32)

    q_in = jnp.concatenate([
        urows_ref[...], lut(g_ref, gt_ref), lut(a_ref, at_ref),
        lut(c_ref, ct_ref), lut(o_ref, ot_ref)], axis=1)
    qe_ref[...] = jnp.dot(q_in, wq_ref[...],
                          preferred_element_type=jnp.float32) + bq_ref[...]

    tv = tvb_ref[...]
    cnt = jnp.maximum(jnp.sum((tv != 0).astype(jnp.float32), axis=1,
                              keepdims=True), 1.0)
    gv = gvb_ref[...]
    gcnt = jnp.maximum(jnp.sum((gv != 0).astype(jnp.float32), axis=1,
                               keepdims=True), 1.0)
    c_in = jnp.concatenate([
        mrows_ref[...], tsb_ref[...] / cnt, gsb_ref[...] / gcnt], axis=1)
    ce_ref[...] = jnp.dot(c_in, wc_ref[...],
                          preferred_element_type=jnp.float32) + bc_ref[...]


def _scores_kernel(q_ref, mt_ref, ts_ref, gs_ref, tv_ref, gv_ref,
                   wc_ref, bc_ref, o_ref):
    j = pl.program_id(0)
    tv = tv_ref[...]
    cnt = jnp.maximum(jnp.sum((tv != 0).astype(jnp.float32), axis=1,
                              keepdims=True), 1.0)
    gv = gv_ref[...]
    gcnt = jnp.maximum(jnp.sum((gv != 0).astype(jnp.float32), axis=1,
                               keepdims=True), 1.0)
    a_in = jnp.concatenate([
        mt_ref[...], ts_ref[...] / cnt, gs_ref[...] / gcnt], axis=1)
    cand = jnp.dot(a_in, wc_ref[...],
                   preferred_element_type=jnp.float32) + bc_ref[...]
    s = lax.dot_general(q_ref[...], cand, (((1,), (1,)), ((), ())),
                        preferred_element_type=jnp.float32)
    col = j * 512 + lax.broadcasted_iota(jnp.int32, s.shape, 1)
    o_ref[...] = jnp.where(col < MOVIES, s, _NEG)


def kernel(user_id, user_gender, raw_user_age, user_gender_X_raw_user_age,
           user_occupation_label, movie_id, movie_title_vector, movie_genres,
           user_table, gender_table, age_table, cross_table, occ_table,
           Wq, bq, movie_table, title_table, genre_table, Wc, bc,
           movies_title_vector, movies_genres):
    f32 = jnp.float32
    i32 = jnp.int32
    title_aug = jnp.concatenate([title_table, jnp.zeros((1, EMB), f32)], 0)
    genre_aug = jnp.concatenate([genre_table, jnp.zeros((1, EMB), f32)], 0)
    tids_all = jnp.concatenate([
        movies_title_vector, jnp.zeros((_COLS - MOVIES, 16), i32),
        movie_title_vector], 0).reshape(-1)
    gids_all = jnp.concatenate([
        movies_genres, jnp.zeros((_COLS - MOVIES, 6), i32),
        movie_genres], 0).reshape(-1)

    tsums, gsums, urows, mrows = _sc_gather(
        title_aug, tids_all, genre_aug, gids_all,
        user_table, user_id, movie_table, movie_id)

    def pad32(t):
        return jnp.zeros((32, EMB), f32).at[: t.shape[0]].set(t)

    qe, ce = pl.pallas_call(
        _towers_kernel,
        out_shape=(jax.ShapeDtypeStruct((B, EMB), f32),
                   jax.ShapeDtypeStruct((B, EMB), f32)),
    )(urows, user_gender[:, None], raw_user_age[:, None],
      user_gender_X_raw_user_age[:, None], user_occupation_label[:, None],
      pad32(gender_table), pad32(age_table), pad32(cross_table),
      pad32(occ_table), Wq, bq[None, :],
      mrows, tsums[_COLS:], gsums[_COLS:], movie_title_vector, movie_genres,
      Wc, bc[None, :])

    mt_pad = jnp.concatenate(
        [movie_table, jnp.zeros((_COLS - MOVIES, EMB), f32)], 0)
    tv_pad = jnp.concatenate(
        [movies_title_vector, jnp.ones((_COLS - MOVIES, 16), i32)], 0)
    gv_pad = jnp.concatenate(
        [movies_genres, jnp.ones((_COLS - MOVIES, 6), i32)], 0)

    scores = pl.pallas_call(
        _scores_kernel,
        grid=(_COLS // 512,),
        in_specs=[
            pl.BlockSpec((B, EMB), lambda j: (0, 0)),
            pl.BlockSpec((512, EMB), lambda j: (j, 0)),
            pl.BlockSpec((512, EMB), lambda j: (j, 0)),
            pl.BlockSpec((512, EMB), lambda j: (j, 0)),
            pl.BlockSpec((512, 16), lambda j: (j, 0)),
            pl.BlockSpec((512, 6), lambda j: (j, 0)),
            pl.BlockSpec((96, EMB), lambda j: (0, 0)),
            pl.BlockSpec((1, EMB), lambda j: (0, 0)),
        ],
        out_specs=pl.BlockSpec((B, 512), lambda j: (0, j)),
        out_shape=jax.ShapeDtypeStruct((B, _COLS), f32),
    )(qe, mt_pad, tsums[:_COLS], gsums[:_COLS], tv_pad, gv_pad,
      Wc, bc[None, :])

    _, predictions = lax.top_k(scores, K)
    return (qe, ce, predictions.astype(jnp.int32))
